# Initial kernel scaffold; baseline (speedup 1.0000x reference)
import jax
import jax.numpy as jnp
from jax.experimental import pallas as pl


def kernel(reprs, w1, b1, w2, b2, w3t, b3, x_id):
    raise NotImplementedError("write your pallas kernel here")


# speedup vs baseline: 1.5530x; 1.5530x over previous
import jax
import jax.numpy as jnp
from jax.experimental import pallas as pl
from jax.experimental.pallas import tpu as pltpu

LANE = 128


def _rup(v, m):
    return ((v + m - 1) // m) * m


def _choose_tile(B):
    if B <= LANE:
        return LANE, LANE
    TM = min(2048, max(LANE, _rup(B, 2 * LANE) // 2))
    return TM, _rup(B, TM)


def _dec_kernel(ids_ref,
                tab_ref,
                w1_ref, b1_ref, w2_ref, b2_ref, w3t_ref, b3r_ref,
                out_ref,
                tile_ref):
    TM = out_ref.shape[0]
    P = tile_ref.shape[0] // (TM + 1)
    S = TM + 1
    base = pl.program_id(0) * TM

    for mi in range(TM):
        i0 = pl.multiple_of(ids_ref[0, base + mi], P)
        i1 = pl.multiple_of(ids_ref[1, base + mi], P)
        slab = tab_ref[pl.ds(i0, P), :] + tab_ref[pl.ds(i1, P), :]
        tile_ref[mi:mi + P * S:S, :] = slab

    w1 = w1_ref[...].astype(jnp.bfloat16)
    acc = b1_ref[...]
    for j in range(P):
        xj = tile_ref[pl.ds(j * S, TM), :].astype(jnp.bfloat16)
        acc = acc + jnp.dot(xj, w1[j * LANE:(j + 1) * LANE, :],
                            preferred_element_type=jnp.float32)
    h1 = jnp.tanh(acc)

    h2 = jnp.tanh(
        jnp.dot(h1.astype(jnp.bfloat16), w2_ref[...].astype(jnp.bfloat16),
                preferred_element_type=jnp.float32) + b2_ref[...])

    out = jax.lax.dot_general(
        h2.astype(jnp.bfloat16), w3t_ref[...].astype(jnp.bfloat16),
        dimension_numbers=(((1,), (1,)), ((), ())),
        preferred_element_type=jnp.float32)
    out_ref[...] = out + b3r_ref[...]


def kernel(reprs, w1, b1, w2, b2, w3t, b3, x_id):
    NR, D = reprs.shape
    H = w2.shape[0]
    O = w3t.shape[0]
    P = D // LANE
    B = x_id.shape[0]
    TM, B_pad = _choose_tile(B)
    S = TM + 1

    tab = reprs.reshape(NR * P, LANE)
    ids = jnp.zeros((2, B_pad), jnp.int32).at[:, :B].set(
        x_id.astype(jnp.int32).T * P)
    b3r = b3.reshape(1, O)

    pinned = lambda shp: pl.BlockSpec(shp, lambda i, *_: (0, 0))
    out = pl.pallas_call(
        _dec_kernel,
        out_shape=jax.ShapeDtypeStruct((B_pad, O), jnp.float32),
        grid_spec=pltpu.PrefetchScalarGridSpec(
            num_scalar_prefetch=1,
            grid=(B_pad // TM,),
            in_specs=[
                pinned((NR * P, LANE)),
                pinned((D, H)), pinned((1, H)),
                pinned((H, H)), pinned((1, H)),
                pinned((O, H)), pinned((1, O)),
            ],
            out_specs=pl.BlockSpec((TM, O), lambda i, *_: (i, 0)),
            scratch_shapes=[pltpu.VMEM((P * S, LANE), jnp.float32)],
        ),
        compiler_params=pltpu.CompilerParams(
            dimension_semantics=("parallel",)),
    )(ids, tab, w1, b1, w2, b2, w3t, b3r)
    return out[:B]
